# baseline (speedup 1.0000x reference)
import functools

import jax
import jax.numpy as jnp
from jax import lax
from jax.experimental import pallas as pl
from jax.experimental.pallas import tpu as pltpu
from jax.experimental.pallas import tpu_sc as plsc

_NE = 1000000
_NI = 16384
_NJ = 50
_D = 64
_B = _NI * _NJ
_NW = 32
_G = 128
_NIG = _NI // _G
_TPW = _NIG // _NW
_NBLK = _TPW * _NJ
_GP = _G + 8

_mesh = plsc.VectorSubcoreMesh(core_axis_name="c", subcore_axis_name="s")


@functools.partial(
    pl.kernel,
    mesh=_mesh,
    compiler_params=pltpu.CompilerParams(
        use_tc_tiling_on_sc=False, needs_layout_passes=False
    ),
    out_type=jax.ShapeDtypeStruct((_NJ, _D // 8, _NIG, 8, _G), jnp.float32),
    scratch_types=[
        pltpu.VMEM((_TPW * _G * _NJ,), jnp.int32),
        pltpu.VMEM((4, _G), jnp.int32),
        pltpu.VMEM((4, _G + 16), jnp.int32),
        pltpu.VMEM((4, _G, 2 * _D), jnp.float32),
        pltpu.VMEM((2, _D, _GP), jnp.float32),
        pltpu.SemaphoreType.DMA,
        pltpu.SemaphoreType.DMA,
        pltpu.SemaphoreType.DMA,
        pltpu.SemaphoreType.DMA,
        pltpu.SemaphoreType.DMA,
        pltpu.SemaphoreType.DMA,
    ],
)
def _emb_lookup(idx_hbm, table_hbm, out_hbm, xblk, idxb, pofs, gbuf, tbuf,
                gsem0, gsem1, gsem2, gsem3, wsem0, wsem1):
    wid = lax.axis_index("s") * 2 + lax.axis_index("c")
    pltpu.sync_copy(idx_hbm.at[pl.ds(wid * (_TPW * _G * _NJ), _TPW * _G * _NJ)],
                    xblk)
    gsems = (gsem0, gsem1, gsem2, gsem3)
    wsems = (wsem0, wsem1)
    iota16 = lax.iota(jnp.int32, 16)
    iota_nj = iota16 * _NJ
    zeros16 = iota16 * 0
    rows8 = [iota16 + g * 16 for g in range(8)]

    def prep_and_fire(c, b):
        t = c // _NJ
        j = c - t * _NJ
        base = t * (_G * _NJ) + j
        for g in range(8):
            v = plsc.load_gather(xblk, [iota_nj + (base + g * 16 * _NJ)])
            idxb[b, pl.ds(g * 16, 16)] = lax.shift_right_logical(v, 1)
            pofs[b, pl.ds(g * 16, 16)] = lax.shift_left(v & 1, 6)
        pltpu.async_copy(table_hbm.at[idxb.at[b]], gbuf.at[b], gsems[b])

    def maybe_prefetch(c, b):
        @pl.when(c < _NBLK)
        def _():
            prep_and_fire(c, b)

    def drain_gather(b):
        pltpu.make_async_copy(
            table_hbm.at[pl.ds(0, _G)], gbuf.at[b], gsems[b]
        ).wait()

    def transpose(b, tb_i):
        @plsc.parallel_loop(0, _G, 1, unroll=4)
        def tr(r):
            rvec = zeros16 + r
            off = pofs[b, pl.ds(r, 16)][0]
            for k in range(4):
                v = gbuf[b, r, pl.ds(off + k * 16, 16)]
                plsc.store_scatter(tbuf.at[tb_i], [rows8[k], rvec], v)

    def fire_write(c, tb_i):
        t = c // _NJ
        j = c - t * _NJ
        ig = wid * _TPW + t

        def wr(d, _):
            dt = d // 8
            pltpu.async_copy(
                tbuf.at[tb_i, d, pl.ds(0, _G)],
                out_hbm.at[j, dt, ig, d - dt * 8],
                wsems[tb_i],
            )
            return ()

        lax.fori_loop(0, _D, wr, (), unroll=8)

    def drain_write(tb_i):
        pltpu.make_async_copy(
            table_hbm.at[pl.ds(0, _D)], gbuf.at[0, pl.ds(0, _D)], wsems[tb_i]
        ).wait()

    def maybe_drain_write(c, tb_i):
        @pl.when(c >= 2)
        def _():
            drain_write(tb_i)

    prep_and_fire(0, 0)
    prep_and_fire(1, 1)
    prep_and_fire(2, 2)

    def body(cc, _):
        c0 = cc * 4
        for b in range(4):
            c = c0 + b
            maybe_prefetch(c + 3, (b + 3) % 4)
            drain_gather(b)
            maybe_drain_write(c, b % 2)
            transpose(b, b % 2)
            fire_write(c, b % 2)
        return ()

    lax.fori_loop(0, _NBLK // 4, body, (), unroll=False)
    drain_write(0)
    drain_write(1)


def kernel(x, weight):
    idx = x.reshape(_B).astype(jnp.int32)
    wt = weight.reshape(_NE // 2, 2 * _D)
    out5 = _emb_lookup(idx, wt)
    return out5.transpose(2, 4, 0, 1, 3).reshape(_NI, _NJ, _D)


# speedup vs baseline: 1.1117x; 1.1117x over previous
import functools

import jax
import jax.numpy as jnp
from jax import lax
from jax.experimental import pallas as pl
from jax.experimental.pallas import tpu as pltpu
from jax.experimental.pallas import tpu_sc as plsc

_NI = 16384
_NJ = 50
_D = 64
_B = _NI * _NJ
_NW = 32
_G = 128
_NIG = _NI // _G
_TPW = _NIG // _NW
_NBLK = _TPW * _NJ
_GP = _G + 8

_mesh = plsc.VectorSubcoreMesh(core_axis_name="c", subcore_axis_name="s")


@functools.partial(
    pl.kernel,
    mesh=_mesh,
    compiler_params=pltpu.CompilerParams(
        use_tc_tiling_on_sc=False, needs_layout_passes=False
    ),
    out_type=jax.ShapeDtypeStruct((_NJ, _D // 8, _NIG, 8, _G), jnp.float32),
    scratch_types=[
        pltpu.VMEM((_TPW * _G * _NJ,), jnp.int32),
        pltpu.VMEM((4, _G), jnp.int32),
        pltpu.VMEM((4, _G, _D), jnp.float32),
        pltpu.VMEM((2, _D, _GP), jnp.float32),
        pltpu.SemaphoreType.DMA,
        pltpu.SemaphoreType.DMA,
        pltpu.SemaphoreType.DMA,
        pltpu.SemaphoreType.DMA,
        pltpu.SemaphoreType.DMA,
        pltpu.SemaphoreType.DMA,
    ],
)
def _emb_lookup(idx_hbm, table_hbm, out_hbm, xblk, idxb, gbuf, tbuf,
                gsem0, gsem1, gsem2, gsem3, wsem0, wsem1):
    wid = lax.axis_index("s") * 2 + lax.axis_index("c")
    pltpu.sync_copy(idx_hbm.at[pl.ds(wid * (_TPW * _G * _NJ), _TPW * _G * _NJ)],
                    xblk)
    gsems = (gsem0, gsem1, gsem2, gsem3)
    wsems = (wsem0, wsem1)
    iota16 = lax.iota(jnp.int32, 16)
    iota_nj = iota16 * _NJ
    zeros16 = iota16 * 0
    rows8 = [iota16 + g * 16 for g in range(8)]

    def prep_and_fire(c, b):
        t = c // _NJ
        j = c - t * _NJ
        base = t * (_G * _NJ) + j
        for g in range(8):
            v = plsc.load_gather(xblk, [iota_nj + (base + g * 16 * _NJ)])
            idxb[b, pl.ds(g * 16, 16)] = v
        pltpu.async_copy(table_hbm.at[idxb.at[b]], gbuf.at[b], gsems[b])

    def maybe_prefetch(c, b):
        @pl.when(c < _NBLK)
        def _():
            prep_and_fire(c, b)

    def drain_gather(b):
        pltpu.make_async_copy(
            table_hbm.at[pl.ds(0, _G)], gbuf.at[b], gsems[b]
        ).wait()

    def transpose(b, tb_i):
        @plsc.parallel_loop(0, _G, 1, unroll=4)
        def tr(r):
            rvec = zeros16 + r
            for k in range(4):
                v = gbuf[b, r, pl.ds(k * 16, 16)]
                plsc.store_scatter(tbuf.at[tb_i], [rows8[k], rvec], v)

    def fire_write(c, tb_i):
        t = c // _NJ
        j = c - t * _NJ
        ig = wid * _TPW + t

        def wr(d, _):
            dt = d // 8
            pltpu.async_copy(
                tbuf.at[tb_i, d, pl.ds(0, _G)],
                out_hbm.at[j, dt, ig, d - dt * 8],
                wsems[tb_i],
            )
            return ()

        lax.fori_loop(0, _D, wr, (), unroll=8)

    def drain_write(tb_i):
        pltpu.make_async_copy(
            table_hbm.at[pl.ds(0, _G)], gbuf.at[0], wsems[tb_i]
        ).wait()

    def maybe_drain_write(c, tb_i):
        @pl.when(c >= 2)
        def _():
            drain_write(tb_i)

    prep_and_fire(0, 0)
    prep_and_fire(1, 1)
    prep_and_fire(2, 2)

    def body(cc, _):
        c0 = cc * 4
        for b in range(4):
            c = c0 + b
            maybe_prefetch(c + 3, (b + 3) % 4)
            drain_gather(b)
            maybe_drain_write(c, b % 2)
            transpose(b, b % 2)
            fire_write(c, b % 2)
        return ()

    lax.fori_loop(0, _NBLK // 4, body, (), unroll=False)
    drain_write(0)
    drain_write(1)


def kernel(x, weight):
    idx = x.reshape(_B).astype(jnp.int32)
    out5 = _emb_lookup(idx, weight)
    return out5.transpose(2, 4, 0, 1, 3).reshape(_NI, _NJ, _D)
